# Initial kernel scaffold; baseline (speedup 1.0000x reference)
import jax
import jax.numpy as jnp
from jax.experimental import pallas as pl


def kernel(x, batch, filt, W, b):
    raise NotImplementedError("write your pallas kernel here")


# speedup vs baseline: 6.6352x; 6.6352x over previous
import jax
import jax.numpy as jnp
from jax import lax
from jax.experimental import pallas as pl
from jax.experimental.pallas import tpu as pltpu

N_NODES = 100000
N_FEAT = 128
N_GRAPHS = 64
BS = 2000
NB = N_NODES // BS


def _body(batch_ref, x_ref, filt_ref, w_ref, b_ref, out_ref, acc_ref):
    i = pl.program_id(0)
    seg = batch_ref[0]
    oh = (lax.broadcasted_iota(jnp.int32, (N_GRAPHS, BS), 0) == seg).astype(
        jnp.float32
    )
    part = jnp.dot(
        oh, x_ref[...],
        preferred_element_type=jnp.float32,
        precision=lax.Precision.HIGHEST,
    )

    @pl.when(i == 0)
    def _init():
        acc_ref[...] = part

    @pl.when(i != 0)
    def _acc():
        acc_ref[...] += part

    @pl.when(i == NB - 1)
    def _final():
        sx = acc_ref[...] * filt_ref[...]
        y = lax.dot_general(
            sx, w_ref[...], (((1,), (1,)), ((), ())),
            preferred_element_type=jnp.float32,
            precision=lax.Precision.HIGHEST,
        )
        out_ref[...] = jnp.tanh(y + b_ref[...])


def kernel(x, batch, filt, W, b):
    batch3 = batch.astype(jnp.int32).reshape(NB, 1, BS)
    return pl.pallas_call(
        _body,
        grid=(NB,),
        in_specs=[
            pl.BlockSpec((1, 1, BS), lambda i: (i, 0, 0)),
            pl.BlockSpec((BS, N_FEAT), lambda i: (i, 0)),
            pl.BlockSpec((1, N_FEAT), lambda i: (0, 0)),
            pl.BlockSpec((N_FEAT, N_FEAT), lambda i: (0, 0)),
            pl.BlockSpec((1, N_FEAT), lambda i: (0, 0)),
        ],
        out_specs=pl.BlockSpec((N_GRAPHS, N_FEAT), lambda i: (0, 0)),
        out_shape=jax.ShapeDtypeStruct((N_GRAPHS, N_FEAT), jnp.float32),
        scratch_shapes=[pltpu.VMEM((N_GRAPHS, N_FEAT), jnp.float32)],
    )(batch3, x, filt.reshape(1, N_FEAT), W, b.reshape(1, N_FEAT))
